# baseline (speedup 1.0000x reference)
import functools

import jax
import jax.numpy as jnp
from jax import lax
from jax.experimental import pallas as pl
from jax.experimental.pallas import tpu as pltpu
from jax.experimental.pallas import tpu_sc as plsc

VOCAB = 1_000_000
EMBED = 64
CTX = 200
BV = 8000
NB = VOCAB // BV


def _gather_sum_sc(idx, emb_table):
  mesh = plsc.VectorSubcoreMesh(core_axis_name="c", subcore_axis_name="s")

  @functools.partial(
      pl.kernel,
      mesh=mesh,
      out_type=jax.ShapeDtypeStruct((1, EMBED), jnp.float32),
      compiler_params=pltpu.CompilerParams(use_tc_tiling_on_sc=False),
      scratch_types=[
          pltpu.VMEM((CTX,), jnp.int32),
          pltpu.VMEM((CTX, EMBED), jnp.float32),
          pltpu.VMEM((EMBED,), jnp.float32),
          pltpu.SemaphoreType.DMA,
      ],
  )
  def k(idx_hbm, table_hbm, out_hbm, idx_v, rows_v, sum_v, sem):
    cid = lax.axis_index("c")
    sid = lax.axis_index("s")

    @pl.when(jnp.logical_and(cid == 0, sid == 0))
    def _():
      pltpu.sync_copy(idx_hbm, idx_v)
      pltpu.async_copy(table_hbm.at[idx_v], rows_v, sem).wait()
      zero = jnp.zeros((16,), jnp.float32)

      def body(r, acc):
        return tuple(acc[j] + rows_v[r, pl.ds(16 * j, 16)] for j in range(4))

      acc = lax.fori_loop(0, CTX, body, (zero, zero, zero, zero))
      for j in range(4):
        sum_v[pl.ds(16 * j, 16)] = acc[j]
      pltpu.sync_copy(sum_v, out_hbm.at[0])

  return k(idx, emb_table)


def _proj_body(s_ref, w_ref, b_ref, logits_ref, lse_ref, m_ref, acc_ref):
  i = pl.program_id(0)

  @pl.when(i == 0)
  def _():
    m_ref[0] = jnp.float32(-jnp.inf)
    acc_ref[0] = jnp.float32(0.0)

  logits = lax.dot_general(
      s_ref[...], w_ref[...],
      dimension_numbers=(((1,), (1,)), ((), ())),
      preferred_element_type=jnp.float32) + b_ref[0]
  logits_ref[0] = logits
  m_old = m_ref[0]
  m_new = jnp.maximum(m_old, jnp.max(logits))
  acc_ref[0] = acc_ref[0] * jnp.exp(m_old - m_new) + jnp.sum(
      jnp.exp(logits - m_new))
  m_ref[0] = m_new

  @pl.when(i == NB - 1)
  def _():
    lse_ref[0, 0] = m_new + jnp.log(acc_ref[0])


def _sub_body(logits_ref, lse_ref, out_ref):
  out_ref[...] = logits_ref[...] - lse_ref[0, 0]


def kernel(inputs, emb_table, W, b):
  idx = inputs.astype(jnp.int32)
  summed = _gather_sum_sc(idx, emb_table)
  b3 = b.reshape(NB, 1, BV)
  logits, lse = pl.pallas_call(
      _proj_body,
      grid=(NB,),
      in_specs=[
          pl.BlockSpec((1, EMBED), lambda i: (0, 0)),
          pl.BlockSpec((BV, EMBED), lambda i: (i, 0)),
          pl.BlockSpec((1, 1, BV), lambda i: (i, 0, 0)),
      ],
      out_specs=[
          pl.BlockSpec((1, 1, BV), lambda i: (i, 0, 0)),
          pl.BlockSpec(memory_space=pltpu.SMEM),
      ],
      out_shape=[
          jax.ShapeDtypeStruct((NB, 1, BV), jnp.float32),
          jax.ShapeDtypeStruct((1, 1), jnp.float32),
      ],
      scratch_shapes=[
          pltpu.SMEM((1,), jnp.float32),
          pltpu.SMEM((1,), jnp.float32),
      ],
  )(summed, W, b3)
  out = pl.pallas_call(
      _sub_body,
      grid=(NB,),
      in_specs=[
          pl.BlockSpec((1, 1, BV), lambda i: (i, 0, 0)),
          pl.BlockSpec(memory_space=pltpu.SMEM),
      ],
      out_specs=pl.BlockSpec((1, 1, BV), lambda i: (i, 0, 0)),
      out_shape=jax.ShapeDtypeStruct((NB, 1, BV), jnp.float32),
  )(logits, lse)
  return out.reshape(1, VOCAB)


# speedup vs baseline: 7.8592x; 7.8592x over previous
import functools

import jax
import jax.numpy as jnp
from jax import lax
from jax.experimental import pallas as pl
from jax.experimental.pallas import tpu as pltpu
from jax.experimental.pallas import tpu_sc as plsc

VOCAB = 1_000_000
EMBED = 64
CTX = 200
NW = 32
BVL = 16384
NG = (VOCAB + BVL - 1) // BVL


def _gather_sum_sc(idx, et):
  mesh = plsc.VectorSubcoreMesh(core_axis_name="c", subcore_axis_name="s")

  @functools.partial(
      pl.kernel,
      mesh=mesh,
      out_type=jax.ShapeDtypeStruct((NW, EMBED), jnp.float32),
      compiler_params=pltpu.CompilerParams(needs_layout_passes=False),
      scratch_types=[
          pltpu.VMEM((CTX + 24,), jnp.int32),
          pltpu.VMEM((EMBED, 128), jnp.float32),
          pltpu.VMEM((EMBED,), jnp.float32),
          pltpu.SemaphoreType.DMA,
      ],
  )
  def k(idx_hbm, et_hbm, p_hbm, idx_v, blk_v, acc_v, sem):
    cid = lax.axis_index("c")
    sid = lax.axis_index("s")
    w = sid * 2 + cid
    pltpu.sync_copy(idx_hbm, idx_v.at[pl.ds(0, CTX)])
    zero = jnp.zeros((16,), jnp.float32)
    rows = [lax.iota(jnp.int32, 16) + 16 * g for g in range(4)]
    for g in range(4):
      acc_v[pl.ds(16 * g, 16)] = zero

    @pl.when(w < CTX // 8)
    def _():
      vec = idx_v[pl.ds(8 * w, 16)]
      acc = [zero, zero, zero, zero]
      for e in range(8):
        i = vec[e]
        c = lax.div(i, 128)
        l = lax.rem(i, 128)
        pltpu.sync_copy(et_hbm.at[:, pl.ds(c * 128, 128)], blk_v)
        col = jnp.full((16,), l, jnp.int32)
        for g in range(4):
          acc[g] = acc[g] + plsc.load_gather(blk_v, [rows[g], col])
      for g in range(4):
        acc_v[pl.ds(16 * g, 16)] = acc[g]

    pltpu.sync_copy(acc_v, p_hbm.at[w])

  return k(idx, et)


def _proj_body(p_ref, wt_ref, b_ref, logits_ref, lse_ref, m_ref, acc_ref):
  i = pl.program_id(0)

  @pl.when(i == 0)
  def _():
    m_ref[0] = jnp.float32(-jnp.inf)
    acc_ref[0] = jnp.float32(0.0)

  s = jnp.sum(p_ref[...], axis=0, keepdims=True)
  raw = lax.dot_general(
      s, wt_ref[...],
      dimension_numbers=(((1,), (0,)), ((), ())),
      preferred_element_type=jnp.float32) + b_ref[...].reshape(1, BVL)
  col = i * BVL + lax.broadcasted_iota(jnp.int32, (1, BVL), 1)
  logits = jnp.where(col < VOCAB, raw, jnp.float32(-jnp.inf))
  logits_ref[...] = logits
  m_old = m_ref[0]
  m_new = jnp.maximum(m_old, jnp.max(logits))
  acc_ref[0] = acc_ref[0] * jnp.exp(m_old - m_new) + jnp.sum(
      jnp.exp(logits - m_new))
  m_ref[0] = m_new

  @pl.when(i == NG - 1)
  def _():
    lse_ref[0, 0] = m_new + jnp.log(acc_ref[0])


def _sub_body(logits_ref, lse_ref, out_ref):
  out_ref[...] = logits_ref[...] - lse_ref[0, 0]


def kernel(inputs, emb_table, W, b):
  idx = inputs.astype(jnp.int32)
  et = emb_table.T
  wt = W.T
  partials = _gather_sum_sc(idx, et)
  logits, lse = pl.pallas_call(
      _proj_body,
      grid=(NG,),
      in_specs=[
          pl.BlockSpec((NW, EMBED), lambda i: (0, 0)),
          pl.BlockSpec((EMBED, BVL), lambda i: (0, i)),
          pl.BlockSpec((BVL,), lambda i: (i,)),
      ],
      out_specs=[
          pl.BlockSpec((1, BVL), lambda i: (0, i)),
          pl.BlockSpec(memory_space=pltpu.SMEM),
      ],
      out_shape=[
          jax.ShapeDtypeStruct((1, VOCAB), jnp.float32),
          jax.ShapeDtypeStruct((1, 1), jnp.float32),
      ],
      scratch_shapes=[
          pltpu.SMEM((1,), jnp.float32),
          pltpu.SMEM((1,), jnp.float32),
      ],
  )(partials, wt, b)
  out = pl.pallas_call(
      _sub_body,
      grid=(NG,),
      in_specs=[
          pl.BlockSpec((1, BVL), lambda i: (0, i)),
          pl.BlockSpec(memory_space=pltpu.SMEM),
      ],
      out_specs=pl.BlockSpec((1, BVL), lambda i: (0, i)),
      out_shape=jax.ShapeDtypeStruct((1, VOCAB), jnp.float32),
  )(logits, lse)
  return out


# speedup vs baseline: 10.0166x; 1.2745x over previous
import functools

import jax
import jax.numpy as jnp
from jax import lax
from jax.experimental import pallas as pl
from jax.experimental.pallas import tpu as pltpu
from jax.experimental.pallas import tpu_sc as plsc

VOCAB = 1_000_000
EMBED = 64
CTX = 200
NW = 32
BVL = 16384
NG = (VOCAB + BVL - 1) // BVL
REM = VOCAB - (NG - 1) * BVL


def _gather_sum_sc(idx, et):
  mesh = plsc.VectorSubcoreMesh(core_axis_name="c", subcore_axis_name="s")

  @functools.partial(
      pl.kernel,
      mesh=mesh,
      out_type=jax.ShapeDtypeStruct((NW, EMBED), jnp.float32),
      compiler_params=pltpu.CompilerParams(needs_layout_passes=False),
      scratch_types=[
          pltpu.VMEM((CTX + 24,), jnp.int32),
          pltpu.VMEM((8, EMBED, 128), jnp.float32),
          pltpu.VMEM((EMBED,), jnp.float32),
          pltpu.SemaphoreType.DMA,
      ],
  )
  def k(idx_hbm, et_hbm, p_hbm, idx_v, blk_v, acc_v, sem):
    cid = lax.axis_index("c")
    sid = lax.axis_index("s")
    w = sid * 2 + cid
    pltpu.sync_copy(idx_hbm, idx_v.at[pl.ds(0, CTX)])
    zero = jnp.zeros((16,), jnp.float32)
    rows = [lax.iota(jnp.int32, 16) + 16 * g for g in range(4)]
    for g in range(4):
      acc_v[pl.ds(16 * g, 16)] = zero

    @pl.when(w < CTX // 8)
    def _():
      vec = idx_v[pl.ds(8 * w, 16)]
      copies = []
      for e in range(8):
        c = lax.div(vec[e], 128)
        copies.append(pltpu.async_copy(
            et_hbm.at[:, pl.ds(c * 128, 128)], blk_v.at[e], sem))
      acc = [zero, zero, zero, zero]
      for e in range(8):
        copies[e].wait()
        col = jnp.full((16,), lax.rem(vec[e], 128), jnp.int32)
        buf = jnp.full((16,), e, jnp.int32)
        for g in range(4):
          acc[g] = acc[g] + plsc.load_gather(blk_v, [buf, rows[g], col])
      for g in range(4):
        acc_v[pl.ds(16 * g, 16)] = acc[g]

    pltpu.sync_copy(acc_v, p_hbm.at[w])

  return k(idx, et)


def _proj_body(p_ref, wt_ref, b_ref, out_ref, m_ref, acc_ref):
  i = pl.program_id(0)

  @pl.when(i == 0)
  def _():
    m_ref[0] = jnp.float32(-jnp.inf)
    acc_ref[0] = jnp.float32(0.0)

  s = jnp.sum(p_ref[...], axis=0, keepdims=True)
  raw = lax.dot_general(
      s, wt_ref[...],
      dimension_numbers=(((1,), (0,)), ((), ())),
      preferred_element_type=jnp.float32) + b_ref[...].reshape(1, BVL)
  col = i * BVL + lax.broadcasted_iota(jnp.int32, (1, BVL), 1)
  logits = jnp.where(col < VOCAB, raw, jnp.float32(-jnp.inf))

  @pl.when(i < NG - 1)
  def _():
    out_ref[:, pl.ds(i * BVL, BVL)] = logits

  m_old = m_ref[0]
  m_new = jnp.maximum(m_old, jnp.max(logits))
  acc_new = acc_ref[0] * jnp.exp(m_old - m_new) + jnp.sum(
      jnp.exp(logits - m_new))
  m_ref[0] = m_new
  acc_ref[0] = acc_new

  @pl.when(i == NG - 1)
  def _():
    out_ref[:, pl.ds((NG - 1) * BVL, REM)] = logits[:, :REM]
    lse = m_new + jnp.log(acc_new)

    def _sub(j, carry):
      out_ref[:, pl.ds(j * BVL, BVL)] = out_ref[:, pl.ds(j * BVL, BVL)] - lse
      return carry

    lax.fori_loop(0, NG - 1, _sub, 0)
    out_ref[:, pl.ds((NG - 1) * BVL, REM)] = (
        out_ref[:, pl.ds((NG - 1) * BVL, REM)] - lse)


def kernel(inputs, emb_table, W, b):
  idx = inputs.astype(jnp.int32)
  et = emb_table.T
  wt = W.T
  partials = _gather_sum_sc(idx, et)
  out = pl.pallas_call(
      _proj_body,
      grid=(NG,),
      in_specs=[
          pl.BlockSpec((NW, EMBED), lambda i: (0, 0)),
          pl.BlockSpec((EMBED, BVL), lambda i: (0, i)),
          pl.BlockSpec((BVL,), lambda i: (i,)),
      ],
      out_specs=pl.BlockSpec((1, VOCAB), lambda i: (0, 0)),
      out_shape=jax.ShapeDtypeStruct((1, VOCAB), jnp.float32),
      scratch_shapes=[
          pltpu.SMEM((1,), jnp.float32),
          pltpu.SMEM((1,), jnp.float32),
      ],
  )(partials, wt, b)
  return out


# speedup vs baseline: 12.0540x; 1.2034x over previous
import functools

import jax
import jax.numpy as jnp
from jax import lax
from jax.experimental import pallas as pl
from jax.experimental.pallas import tpu as pltpu
from jax.experimental.pallas import tpu_sc as plsc

VOCAB = 1_000_000
EMBED = 64
CTX = 200
NW = 32
BVL = 32768
NG = (VOCAB + BVL - 1) // BVL
REM = VOCAB - (NG - 1) * BVL


def _gather_sum_sc(idx, et):
  mesh = plsc.VectorSubcoreMesh(core_axis_name="c", subcore_axis_name="s")

  @functools.partial(
      pl.kernel,
      mesh=mesh,
      out_type=jax.ShapeDtypeStruct((NW, EMBED), jnp.float32),
      compiler_params=pltpu.CompilerParams(needs_layout_passes=False),
      scratch_types=[
          pltpu.VMEM((CTX + 24,), jnp.int32),
          pltpu.VMEM((8, EMBED, 128), jnp.float32),
          pltpu.VMEM((EMBED,), jnp.float32),
          pltpu.SemaphoreType.DMA,
      ],
  )
  def k(idx_hbm, et_hbm, p_hbm, idx_v, blk_v, acc_v, sem):
    cid = lax.axis_index("c")
    sid = lax.axis_index("s")
    w = sid * 2 + cid
    pltpu.sync_copy(idx_hbm, idx_v.at[pl.ds(0, CTX)])
    zero = jnp.zeros((16,), jnp.float32)
    rows = [lax.iota(jnp.int32, 16) + 16 * g for g in range(4)]
    for g in range(4):
      acc_v[pl.ds(16 * g, 16)] = zero

    @pl.when(w < CTX // 8)
    def _():
      vec = idx_v[pl.ds(8 * w, 16)]
      copies = []
      for e in range(8):
        c = lax.div(vec[e], 128)
        copies.append(pltpu.async_copy(
            et_hbm.at[:, pl.ds(c * 128, 128)], blk_v.at[e], sem))
      acc = [zero, zero, zero, zero]
      for e in range(8):
        copies[e].wait()
        col = jnp.full((16,), lax.rem(vec[e], 128), jnp.int32)
        buf = jnp.full((16,), e, jnp.int32)
        for g in range(4):
          acc[g] = acc[g] + plsc.load_gather(blk_v, [buf, rows[g], col])
      for g in range(4):
        acc_v[pl.ds(16 * g, 16)] = acc[g]

    pltpu.sync_copy(acc_v, p_hbm.at[w])

  return k(idx, et)


def _proj_body(p_ref, wt_ref, b_ref, out_ref, m_ref, acc_ref):
  i = pl.program_id(0)

  @pl.when(i == 0)
  def _():
    m_ref[0] = jnp.float32(-jnp.inf)
    acc_ref[0] = jnp.float32(0.0)

  s = jnp.sum(p_ref[...], axis=0, keepdims=True)
  raw = lax.dot_general(
      s, wt_ref[...],
      dimension_numbers=(((1,), (0,)), ((), ())),
      preferred_element_type=jnp.float32) + b_ref[...].reshape(1, BVL)
  col = i * BVL + lax.broadcasted_iota(jnp.int32, (1, BVL), 1)
  logits = jnp.where(col < VOCAB, raw, jnp.float32(-jnp.inf))

  @pl.when(i < NG - 1)
  def _():
    out_ref[:, pl.ds(i * BVL, BVL)] = logits

  m_old = m_ref[0]
  m_new = jnp.maximum(m_old, jnp.max(logits))
  acc_new = acc_ref[0] * jnp.exp(m_old - m_new) + jnp.sum(
      jnp.exp(logits - m_new))
  m_ref[0] = m_new
  acc_ref[0] = acc_new

  @pl.when(i == NG - 1)
  def _():
    out_ref[:, pl.ds((NG - 1) * BVL, REM)] = logits[:, :REM]
    lse = m_new + jnp.log(acc_new)

    def _sub(j, carry):
      out_ref[:, pl.ds(j * BVL, BVL)] = out_ref[:, pl.ds(j * BVL, BVL)] - lse
      return carry

    lax.fori_loop(0, NG - 1, _sub, 0)
    out_ref[:, pl.ds((NG - 1) * BVL, REM)] = (
        out_ref[:, pl.ds((NG - 1) * BVL, REM)] - lse)


def kernel(inputs, emb_table, W, b):
  idx = inputs.astype(jnp.int32)
  et = emb_table.T
  wt = W.T
  partials = _gather_sum_sc(idx, et)
  out = pl.pallas_call(
      _proj_body,
      grid=(NG,),
      in_specs=[
          pl.BlockSpec((NW, EMBED), lambda i: (0, 0)),
          pl.BlockSpec((EMBED, BVL), lambda i: (0, i)),
          pl.BlockSpec((BVL,), lambda i: (i,)),
      ],
      out_specs=pl.BlockSpec((1, VOCAB), lambda i: (0, 0)),
      out_shape=jax.ShapeDtypeStruct((1, VOCAB), jnp.float32),
      scratch_shapes=[
          pltpu.SMEM((1,), jnp.float32),
          pltpu.SMEM((1,), jnp.float32),
      ],
  )(partials, wt, b)
  return out
